# baseline (speedup 1.0000x reference)
import functools
import math

import jax
import jax.numpy as jnp
from jax import lax
from jax.experimental import pallas as pl
from jax.experimental.pallas import tpu as pltpu
from jax.experimental.pallas import tpu_sc as plsc

_IMG = 224
_P = 16
_C = 3
_V = 30522
_L = 77
_D = 1024
_E = 8
_H = 8
_NL = 2
_NCLS = 10
_DFF = 2048
_G = _IMG // _P
_NPATCH = _G * _G
_S = _NPATCH + _L
_B = 2
_SP = 288
_T = _B * _SP
_DH = _D // _H
_CPP = _C * _P * _P

_GELU_C = 0.7978845608028654
_DFF2 = _DFF // 2

_NW = 32
_BPW = 8
_GROWS = _NW * _BPW


def _ln_f32(x, g, b):
    m = jnp.mean(x, axis=-1, keepdims=True)
    v = jnp.mean((x - m) * (x - m), axis=-1, keepdims=True)
    return (x - m) * lax.rsqrt(v + 1e-5) * g + b


def _gelu(x):
    x3 = x * x * x
    return 0.5 * x * (1.0 + jnp.tanh(_GELU_C * (x + 0.044715 * x3)))


def _sc_gather(table, idx):
    mesh = plsc.VectorSubcoreMesh(core_axis_name="c", subcore_axis_name="s")

    @functools.partial(
        pl.kernel,
        mesh=mesh,
        out_type=jax.ShapeDtypeStruct((_GROWS, _D), jnp.float32),
        scratch_types=[
            pltpu.VMEM((_BPW,), jnp.int32),
            pltpu.VMEM((_BPW, _D), jnp.float32),
            pltpu.SemaphoreType.DMA,
        ],
    )
    def k(table_hbm, idx_hbm, out_hbm, idx_v, rows_v, sem):
        wid = lax.axis_index("s") * 2 + lax.axis_index("c")
        base = wid * _BPW
        pltpu.sync_copy(idx_hbm.at[pl.ds(base, _BPW)], idx_v)
        pltpu.async_copy(table_hbm.at[idx_v], rows_v, sem).wait()
        pltpu.sync_copy(rows_v, out_hbm.at[pl.ds(base, _BPW)])

    return k(table, idx)


def _embed_body(pat_ref, wp_ref, bp_ref, pos_img_ref, pos_txt_ref,
                mod0_ref, mod1_ref, txt_ref, out_ref):
    wp = wp_ref[...]
    zpad = jnp.zeros((_SP - _S, _D), jnp.float32)
    parts = []
    for b in range(_B):
        img = jnp.dot(pat_ref[b * _NPATCH:(b + 1) * _NPATCH, :], wp,
                      preferred_element_type=jnp.float32)
        img = img + bp_ref[...] + pos_img_ref[...] + mod0_ref[...]
        txt = txt_ref[b * _L:(b + 1) * _L, :] + pos_txt_ref[...] + mod1_ref[...]
        parts += [img, txt, zpad]
    out_ref[...] = jnp.concatenate(parts, axis=0)


def _embed(patches, wp, bp, pos_img, pos_txt, mod0, mod1, txt):
    return pl.pallas_call(
        _embed_body,
        out_shape=jax.ShapeDtypeStruct((_T, _D), jnp.float32),
    )(patches, wp, bp, pos_img, pos_txt, mod0, mod1, txt)


def _layer_body(with_head, *refs):
    if with_head:
        (h_ref, mask_ref, g1_ref, b1n_ref, wqkv_ref, bqkv_ref, wo_ref, bo_ref,
         g2_ref, b2n_ref, wr_ref, br_ref, w1_ref, b1_ref, w2_ref, b2_ref,
         gf_ref, bf_ref, wc1_ref, bc1_ref, wc2_ref, bc2_ref,
         out_ref, lg_ref, tln_ref, hmid_ref, gates_ref) = refs
    else:
        (h_ref, mask_ref, g1_ref, b1n_ref, wqkv_ref, bqkv_ref, wo_ref, bo_ref,
         g2_ref, b2n_ref, wr_ref, br_ref, w1_ref, b1_ref, w2_ref, b2_ref,
         out_ref, tln_ref, hmid_ref, gates_ref) = refs
    o_ref = tln_ref
    e = pl.program_id(0)
    j = pl.program_id(1)

    @pl.when((e == 0) & (j == 0))
    def _attn_and_router():
        ln = _ln_f32(h_ref[...], g1_ref[...], b1n_ref[...])
        scale = 1.0 / math.sqrt(_DH)
        for b in range(_B):
            lb = ln[b * _SP:(b + 1) * _SP, :]
            bias = (1.0 - mask_ref[b:b + 1, :]) * (-1e9)
            for hh in range(_H):
                wq = wqkv_ref[:, hh * _DH:(hh + 1) * _DH]
                wk = wqkv_ref[:, _D + hh * _DH:_D + (hh + 1) * _DH]
                wv = wqkv_ref[:, 2 * _D + hh * _DH:2 * _D + (hh + 1) * _DH]
                q = jnp.dot(lb, wq, preferred_element_type=jnp.float32)
                q = q + bqkv_ref[:, hh * _DH:(hh + 1) * _DH]
                k = jnp.dot(lb, wk, preferred_element_type=jnp.float32)
                k = k + bqkv_ref[:, _D + hh * _DH:_D + (hh + 1) * _DH]
                v = jnp.dot(lb, wv, preferred_element_type=jnp.float32)
                v = v + bqkv_ref[:, 2 * _D + hh * _DH:2 * _D + (hh + 1) * _DH]
                att = lax.dot_general(q, k, (((1,), (1,)), ((), ())),
                                      preferred_element_type=jnp.float32)
                att = att * scale + bias
                att = att - jnp.max(att, axis=-1, keepdims=True)
                pr = jnp.exp(att)
                pr = pr / jnp.sum(pr, axis=-1, keepdims=True)
                o_ref[b * _SP:(b + 1) * _SP, hh * _DH:(hh + 1) * _DH] = (
                    jnp.dot(pr, v, preferred_element_type=jnp.float32))
        out = jnp.dot(o_ref[...], wo_ref[...], preferred_element_type=jnp.float32)
        xa = h_ref[...] + out + bo_ref[...]
        out_ref[...] = xa

        ln2 = _ln_f32(xa, g2_ref[...], b2n_ref[...])
        tln_ref[...] = ln2
        logits = jnp.dot(ln2, wr_ref[...], preferred_element_type=jnp.float32)
        logits = logits + br_ref[...]
        lanes = lax.broadcasted_iota(jnp.int32, (_T, _E), 1)
        m1 = jnp.max(logits, axis=-1, keepdims=True)
        a1 = jnp.min(jnp.where(logits == m1, lanes, _E), axis=-1, keepdims=True)
        l2 = jnp.where(lanes == a1, -jnp.inf, logits)
        m2 = jnp.max(l2, axis=-1, keepdims=True)
        a2 = jnp.min(jnp.where(l2 == m2, lanes, _E), axis=-1, keepdims=True)
        e2 = jnp.exp(m2 - m1)
        w1g = 1.0 / (1.0 + e2)
        w2g = e2 / (1.0 + e2)
        gates_ref[...] = (jnp.where(lanes == a1, w1g, 0.0)
                          + jnp.where(lanes == a2, w2g, 0.0))

    @pl.when(j == 0)
    def _expand():
        hmid = jnp.dot(tln_ref[...], w1_ref[0], preferred_element_type=jnp.float32)
        hmid = _gelu(hmid + b1_ref[0])
        hmid_ref[0] = hmid[:, :_DFF2]
        hmid_ref[1] = hmid[:, _DFF2:]

    onehot = (lax.broadcasted_iota(jnp.int32, (_E, 1), 0) == e).astype(jnp.float32)
    gcol = jnp.dot(gates_ref[...], onehot, preferred_element_type=jnp.float32)

    contrib = jnp.dot(hmid_ref[j], w2_ref[0], preferred_element_type=jnp.float32)

    @pl.when(j == 0)
    def _bias2():
        out_ref[...] += gcol * b2_ref[0]

    out_ref[...] += gcol * contrib

    if with_head:
        @pl.when((e == _E - 1) & (j == 1))
        def _do_head():
            lnf = _ln_f32(out_ref[...], gf_ref[...], bf_ref[...])
            riota = lax.broadcasted_iota(jnp.int32, (_SP, 1), 0)
            wrow = jnp.where(riota < _S, 1.0 / _S, 0.0)
            fvs = []
            for b in range(_B):
                fvs.append(jnp.sum(lnf[b * _SP:(b + 1) * _SP, :] * wrow,
                                   axis=0, keepdims=True))
            fv = jnp.concatenate(fvs, axis=0)
            hcl = jnp.dot(fv, wc1_ref[...], preferred_element_type=jnp.float32)
            hcl = jnp.maximum(hcl + bc1_ref[...], 0.0)
            lg = jnp.dot(hcl, wc2_ref[...], preferred_element_type=jnp.float32)
            lg_ref[...] = lg + bc2_ref[...]


def _layer(h, mask, lp, head_params=None):
    const2 = lambda e, j: (0, 0)
    in_specs = [
        pl.BlockSpec((_T, _D), const2),
        pl.BlockSpec((_B, _SP), const2),
        pl.BlockSpec((1, _D), const2),
        pl.BlockSpec((1, _D), const2),
        pl.BlockSpec((_D, 3 * _D), const2),
        pl.BlockSpec((1, 3 * _D), const2),
        pl.BlockSpec((_D, _D), const2),
        pl.BlockSpec((1, _D), const2),
        pl.BlockSpec((1, _D), const2),
        pl.BlockSpec((1, _D), const2),
        pl.BlockSpec((_D, _E), const2),
        pl.BlockSpec((1, _E), const2),
        pl.BlockSpec((1, _D, _DFF), lambda e, j: (e, 0, 0)),
        pl.BlockSpec((1, 1, _DFF), lambda e, j: (e, 0, 0)),
        pl.BlockSpec((1, _DFF2, _D), lambda e, j: (e, j, 0)),
        pl.BlockSpec((1, 1, _D), lambda e, j: (e, 0, 0)),
    ]
    args = [h, mask, lp['g1'].reshape(1, _D), lp['b1n'].reshape(1, _D),
            lp['Wqkv'], lp['bqkv'].reshape(1, 3 * _D), lp['Wo'],
            lp['bo'].reshape(1, _D), lp['g2'].reshape(1, _D),
            lp['b2n'].reshape(1, _D), lp['Wr'], lp['br'].reshape(1, _E),
            lp['W1'], lp['b1'].reshape(_E, 1, _DFF), lp['W2'],
            lp['b2'].reshape(_E, 1, _D)]
    with_head = head_params is not None
    if with_head:
        gf, bf, wc1, bc1, wc2, bc2 = head_params
        in_specs += [
            pl.BlockSpec((1, _D), const2),
            pl.BlockSpec((1, _D), const2),
            pl.BlockSpec((_D, _D // 2), const2),
            pl.BlockSpec((1, _D // 2), const2),
            pl.BlockSpec((_D // 2, _NCLS), const2),
            pl.BlockSpec((1, _NCLS), const2),
        ]
        args += [gf.reshape(1, _D), bf.reshape(1, _D), wc1,
                 bc1.reshape(1, _D // 2), wc2, bc2.reshape(1, _NCLS)]
        out_specs = [pl.BlockSpec((_T, _D), const2),
                     pl.BlockSpec((_B, _NCLS), const2)]
        out_shape = [jax.ShapeDtypeStruct((_T, _D), jnp.float32),
                     jax.ShapeDtypeStruct((_B, _NCLS), jnp.float32)]
    else:
        out_specs = pl.BlockSpec((_T, _D), const2)
        out_shape = jax.ShapeDtypeStruct((_T, _D), jnp.float32)

    return pl.pallas_call(
        functools.partial(_layer_body, with_head),
        grid=(_E, 2),
        in_specs=in_specs,
        out_specs=out_specs,
        out_shape=out_shape,
        scratch_shapes=[
            pltpu.VMEM((_T, _D), jnp.float32),
            pltpu.VMEM((2, _T, _DFF2), jnp.float32),
            pltpu.VMEM((_T, _E), jnp.float32),
        ],
    )(*args)


def _head_body(h_ref, gf_ref, bf_ref, wc1_ref, bc1_ref, wc2_ref, bc2_ref,
               out_ref):
    ln = _ln_f32(h_ref[...], gf_ref[...], bf_ref[...])
    riota = lax.broadcasted_iota(jnp.int32, (_SP, 1), 0)
    w = jnp.where(riota < _S, 1.0 / _S, 0.0)
    fvs = []
    for b in range(_B):
        fvs.append(jnp.sum(ln[b * _SP:(b + 1) * _SP, :] * w, axis=0,
                           keepdims=True))
    fv = jnp.concatenate(fvs, axis=0)
    hcl = jnp.dot(fv, wc1_ref[...], preferred_element_type=jnp.float32)
    hcl = jnp.maximum(hcl + bc1_ref[...], 0.0)
    lg = jnp.dot(hcl, wc2_ref[...], preferred_element_type=jnp.float32)
    out_ref[...] = lg + bc2_ref[...]


def _head(h, gf, bf, wc1, bc1, wc2, bc2):
    return pl.pallas_call(
        _head_body,
        out_shape=jax.ShapeDtypeStruct((_B, _NCLS), jnp.float32),
    )(h, gf, bf, wc1, bc1, wc2, bc2)


def kernel(images, input_ids, attention_mask, params):
    p = params
    patches = images.reshape(_B, _C, _G, _P, _G, _P)
    patches = patches.transpose(0, 2, 4, 1, 3, 5).reshape(_B * _NPATCH, _CPP)

    ids = input_ids.reshape(-1).astype(jnp.int32)
    ids = jnp.concatenate([ids, jnp.zeros((_GROWS - _B * _L,), jnp.int32)])
    txt_rows = _sc_gather(p['tok_emb'], ids)[:_B * _L]

    h = _embed(patches, p['Wp'], p['bp'].reshape(1, _D), p['pos_img'],
               p['pos_txt'], p['mod'][0:1], p['mod'][1:2], txt_rows)

    mask = jnp.concatenate(
        [jnp.ones((_B, _NPATCH), jnp.float32),
         attention_mask.astype(jnp.float32),
         jnp.zeros((_B, _SP - _S), jnp.float32)], axis=1)

    h = _layer(h, mask, p['layers'][0])
    h = _layer(h, mask, p['layers'][1])
    return _head(h, p['gf'].reshape(1, _D), p['bf'].reshape(1, _D),
                 p['Wc1'], p['bc1'].reshape(1, _D // 2),
                 p['Wc2'], p['bc2'].reshape(1, _NCLS))


# speedup vs baseline: 1.2287x; 1.2287x over previous
import functools
import math

import jax
import jax.numpy as jnp
from jax import lax
from jax.experimental import pallas as pl
from jax.experimental.pallas import tpu as pltpu
from jax.experimental.pallas import tpu_sc as plsc

_IMG = 224
_P = 16
_C = 3
_V = 30522
_L = 77
_D = 1024
_E = 8
_H = 8
_NL = 2
_NCLS = 10
_DFF = 2048
_G = _IMG // _P
_NPATCH = _G * _G
_S = _NPATCH + _L
_B = 2
_SP = 288
_T = _B * _SP
_DH = _D // _H
_CPP = _C * _P * _P

_GELU_C = 0.7978845608028654
_DFF2 = _DFF // 2

_NW = 32
_BPW = 8
_GROWS = _NW * _BPW


def _ln_f32(x, g, b):
    m = jnp.mean(x, axis=-1, keepdims=True)
    v = jnp.mean((x - m) * (x - m), axis=-1, keepdims=True)
    return (x - m) * lax.rsqrt(v + 1e-5) * g + b


def _gelu(x):
    x3 = x * x * x
    return 0.5 * x * (1.0 + jnp.tanh(_GELU_C * (x + 0.044715 * x3)))


def _sc_gather(table, idx):
    mesh = plsc.VectorSubcoreMesh(core_axis_name="c", subcore_axis_name="s")

    @functools.partial(
        pl.kernel,
        mesh=mesh,
        out_type=jax.ShapeDtypeStruct((_GROWS, _D), jnp.float32),
        scratch_types=[
            pltpu.VMEM((_BPW,), jnp.int32),
            pltpu.VMEM((_BPW, _D), jnp.float32),
            pltpu.SemaphoreType.DMA,
        ],
    )
    def k(table_hbm, idx_hbm, out_hbm, idx_v, rows_v, sem):
        wid = lax.axis_index("s") * 2 + lax.axis_index("c")
        base = wid * _BPW
        pltpu.sync_copy(idx_hbm.at[pl.ds(base, _BPW)], idx_v)
        pltpu.async_copy(table_hbm.at[idx_v], rows_v, sem).wait()
        pltpu.sync_copy(rows_v, out_hbm.at[pl.ds(base, _BPW)])

    return k(table, idx)


def _embed_body(pat_ref, wp_ref, bp_ref, pos_img_ref, pos_txt_ref,
                mod0_ref, mod1_ref, txt_ref, out_ref):
    wp = wp_ref[...]
    zpad = jnp.zeros((_SP - _S, _D), jnp.float32)
    parts = []
    for b in range(_B):
        img = jnp.dot(pat_ref[b * _NPATCH:(b + 1) * _NPATCH, :], wp,
                      preferred_element_type=jnp.float32)
        img = img + bp_ref[...] + pos_img_ref[...] + mod0_ref[...]
        txt = txt_ref[b * _L:(b + 1) * _L, :] + pos_txt_ref[...] + mod1_ref[...]
        parts += [img, txt, zpad]
    out_ref[...] = jnp.concatenate(parts, axis=0)


def _embed(patches, wp, bp, pos_img, pos_txt, mod0, mod1, txt):
    return pl.pallas_call(
        _embed_body,
        out_shape=jax.ShapeDtypeStruct((_T, _D), jnp.float32),
    )(patches, wp, bp, pos_img, pos_txt, mod0, mod1, txt)


def _layer_body(with_head, *refs):
    if with_head:
        (h_ref, mask_ref, g1_ref, b1n_ref, wqkv_ref, bqkv_ref, wo_ref, bo_ref,
         g2_ref, b2n_ref, wr_ref, br_ref, w1_ref, b1_ref, w2_ref, b2_ref,
         gf_ref, bf_ref, wc1_ref, bc1_ref, wc2_ref, bc2_ref,
         out_ref, lg_ref, tln_ref, gates_ref) = refs
    else:
        (h_ref, mask_ref, g1_ref, b1n_ref, wqkv_ref, bqkv_ref, wo_ref, bo_ref,
         g2_ref, b2n_ref, wr_ref, br_ref, w1_ref, b1_ref, w2_ref, b2_ref,
         out_ref, tln_ref, gates_ref) = refs
    o_ref = tln_ref
    e = pl.program_id(0)
    j = pl.program_id(1)

    @pl.when((e == 0) & (j == 0))
    def _attn_and_router():
        ln = _ln_f32(h_ref[...], g1_ref[...], b1n_ref[...])
        scale = 1.0 / math.sqrt(_DH)
        for b in range(_B):
            lb = ln[b * _SP:(b + 1) * _SP, :]
            bias = (1.0 - mask_ref[b:b + 1, :]) * (-1e9)
            for hh in range(_H):
                wq = wqkv_ref[:, hh * _DH:(hh + 1) * _DH]
                wk = wqkv_ref[:, _D + hh * _DH:_D + (hh + 1) * _DH]
                wv = wqkv_ref[:, 2 * _D + hh * _DH:2 * _D + (hh + 1) * _DH]
                q = jnp.dot(lb, wq, preferred_element_type=jnp.float32)
                q = q + bqkv_ref[:, hh * _DH:(hh + 1) * _DH]
                k = jnp.dot(lb, wk, preferred_element_type=jnp.float32)
                k = k + bqkv_ref[:, _D + hh * _DH:_D + (hh + 1) * _DH]
                v = jnp.dot(lb, wv, preferred_element_type=jnp.float32)
                v = v + bqkv_ref[:, 2 * _D + hh * _DH:2 * _D + (hh + 1) * _DH]
                att = lax.dot_general(q, k, (((1,), (1,)), ((), ())),
                                      preferred_element_type=jnp.float32)
                att = att * scale + bias
                att = att - jnp.max(att, axis=-1, keepdims=True)
                pr = jnp.exp(att)
                pr = pr / jnp.sum(pr, axis=-1, keepdims=True)
                o_ref[b * _SP:(b + 1) * _SP, hh * _DH:(hh + 1) * _DH] = (
                    jnp.dot(pr, v, preferred_element_type=jnp.float32))
        out = jnp.dot(o_ref[...], wo_ref[...], preferred_element_type=jnp.float32)
        xa = h_ref[...] + out + bo_ref[...]
        out_ref[...] = xa

        ln2 = _ln_f32(xa, g2_ref[...], b2n_ref[...])
        tln_ref[...] = ln2
        logits = jnp.dot(ln2, wr_ref[...], preferred_element_type=jnp.float32)
        logits = logits + br_ref[...]
        lanes = lax.broadcasted_iota(jnp.int32, (_T, _E), 1)
        m1 = jnp.max(logits, axis=-1, keepdims=True)
        a1 = jnp.min(jnp.where(logits == m1, lanes, _E), axis=-1, keepdims=True)
        l2 = jnp.where(lanes == a1, -jnp.inf, logits)
        m2 = jnp.max(l2, axis=-1, keepdims=True)
        a2 = jnp.min(jnp.where(l2 == m2, lanes, _E), axis=-1, keepdims=True)
        e2 = jnp.exp(m2 - m1)
        w1g = 1.0 / (1.0 + e2)
        w2g = e2 / (1.0 + e2)
        gates_ref[...] = (jnp.where(lanes == a1, w1g, 0.0)
                          + jnp.where(lanes == a2, w2g, 0.0))

    hmid = jnp.dot(tln_ref[...], w1_ref[0], preferred_element_type=jnp.float32)
    hmid = _gelu(hmid + b1_ref[0])
    contrib = jnp.dot(hmid, w2_ref[0], preferred_element_type=jnp.float32)

    onehot = (lax.broadcasted_iota(jnp.int32, (_E, 1), 0) == e).astype(jnp.float32)
    gcol = jnp.dot(gates_ref[...], onehot, preferred_element_type=jnp.float32)

    @pl.when(j == 0)
    def _bias2():
        out_ref[...] += gcol * b2_ref[0]

    out_ref[...] += gcol * contrib

    if with_head:
        @pl.when((e == _E - 1) & (j == 1))
        def _do_head():
            lnf = _ln_f32(out_ref[...], gf_ref[...], bf_ref[...])
            riota = lax.broadcasted_iota(jnp.int32, (_SP, 1), 0)
            wrow = jnp.where(riota < _S, 1.0 / _S, 0.0)
            fvs = []
            for b in range(_B):
                fvs.append(jnp.sum(lnf[b * _SP:(b + 1) * _SP, :] * wrow,
                                   axis=0, keepdims=True))
            fv = jnp.concatenate(fvs, axis=0)
            hcl = jnp.dot(fv, wc1_ref[...], preferred_element_type=jnp.float32)
            hcl = jnp.maximum(hcl + bc1_ref[...], 0.0)
            lg = jnp.dot(hcl, wc2_ref[...], preferred_element_type=jnp.float32)
            lg_ref[...] = lg + bc2_ref[...]


def _layer(h, mask, lp, head_params=None):
    const2 = lambda e, j: (0, 0)
    in_specs = [
        pl.BlockSpec((_T, _D), const2),
        pl.BlockSpec((_B, _SP), const2),
        pl.BlockSpec((1, _D), const2),
        pl.BlockSpec((1, _D), const2),
        pl.BlockSpec((_D, 3 * _D), const2),
        pl.BlockSpec((1, 3 * _D), const2),
        pl.BlockSpec((_D, _D), const2),
        pl.BlockSpec((1, _D), const2),
        pl.BlockSpec((1, _D), const2),
        pl.BlockSpec((1, _D), const2),
        pl.BlockSpec((_D, _E), const2),
        pl.BlockSpec((1, _E), const2),
        pl.BlockSpec((1, _D, _DFF2), lambda e, j: (e, 0, j)),
        pl.BlockSpec((1, 1, _DFF2), lambda e, j: (e, 0, j)),
        pl.BlockSpec((1, _DFF2, _D), lambda e, j: (e, j, 0)),
        pl.BlockSpec((1, 1, _D), lambda e, j: (e, 0, 0)),
    ]
    args = [h, mask, lp['g1'].reshape(1, _D), lp['b1n'].reshape(1, _D),
            lp['Wqkv'], lp['bqkv'].reshape(1, 3 * _D), lp['Wo'],
            lp['bo'].reshape(1, _D), lp['g2'].reshape(1, _D),
            lp['b2n'].reshape(1, _D), lp['Wr'], lp['br'].reshape(1, _E),
            lp['W1'], lp['b1'].reshape(_E, 1, _DFF), lp['W2'],
            lp['b2'].reshape(_E, 1, _D)]
    with_head = head_params is not None
    if with_head:
        gf, bf, wc1, bc1, wc2, bc2 = head_params
        in_specs += [
            pl.BlockSpec((1, _D), const2),
            pl.BlockSpec((1, _D), const2),
            pl.BlockSpec((_D, _D // 2), const2),
            pl.BlockSpec((1, _D // 2), const2),
            pl.BlockSpec((_D // 2, _NCLS), const2),
            pl.BlockSpec((1, _NCLS), const2),
        ]
        args += [gf.reshape(1, _D), bf.reshape(1, _D), wc1,
                 bc1.reshape(1, _D // 2), wc2, bc2.reshape(1, _NCLS)]
        out_specs = [pl.BlockSpec((_T, _D), const2),
                     pl.BlockSpec((_B, _NCLS), const2)]
        out_shape = [jax.ShapeDtypeStruct((_T, _D), jnp.float32),
                     jax.ShapeDtypeStruct((_B, _NCLS), jnp.float32)]
    else:
        out_specs = pl.BlockSpec((_T, _D), const2)
        out_shape = jax.ShapeDtypeStruct((_T, _D), jnp.float32)

    return pl.pallas_call(
        functools.partial(_layer_body, with_head),
        grid=(_E, 2),
        in_specs=in_specs,
        out_specs=out_specs,
        out_shape=out_shape,
        scratch_shapes=[
            pltpu.VMEM((_T, _D), jnp.float32),
            pltpu.VMEM((_T, _E), jnp.float32),
        ],
    )(*args)


def _head_body(h_ref, gf_ref, bf_ref, wc1_ref, bc1_ref, wc2_ref, bc2_ref,
               out_ref):
    ln = _ln_f32(h_ref[...], gf_ref[...], bf_ref[...])
    riota = lax.broadcasted_iota(jnp.int32, (_SP, 1), 0)
    w = jnp.where(riota < _S, 1.0 / _S, 0.0)
    fvs = []
    for b in range(_B):
        fvs.append(jnp.sum(ln[b * _SP:(b + 1) * _SP, :] * w, axis=0,
                           keepdims=True))
    fv = jnp.concatenate(fvs, axis=0)
    hcl = jnp.dot(fv, wc1_ref[...], preferred_element_type=jnp.float32)
    hcl = jnp.maximum(hcl + bc1_ref[...], 0.0)
    lg = jnp.dot(hcl, wc2_ref[...], preferred_element_type=jnp.float32)
    out_ref[...] = lg + bc2_ref[...]


def _head(h, gf, bf, wc1, bc1, wc2, bc2):
    return pl.pallas_call(
        _head_body,
        out_shape=jax.ShapeDtypeStruct((_B, _NCLS), jnp.float32),
    )(h, gf, bf, wc1, bc1, wc2, bc2)


def kernel(images, input_ids, attention_mask, params):
    p = params
    patches = images.reshape(_B, _C, _G, _P, _G, _P)
    patches = patches.transpose(0, 2, 4, 1, 3, 5).reshape(_B * _NPATCH, _CPP)

    ids = input_ids.reshape(-1).astype(jnp.int32)
    ids = jnp.concatenate([ids, jnp.zeros((_GROWS - _B * _L,), jnp.int32)])
    txt_rows = _sc_gather(p['tok_emb'], ids)[:_B * _L]

    h = _embed(patches, p['Wp'], p['bp'].reshape(1, _D), p['pos_img'],
               p['pos_txt'], p['mod'][0:1], p['mod'][1:2], txt_rows)

    mask = jnp.concatenate(
        [jnp.ones((_B, _NPATCH), jnp.float32),
         attention_mask.astype(jnp.float32),
         jnp.zeros((_B, _SP - _S), jnp.float32)], axis=1)

    h = _layer(h, mask, p['layers'][0])
    h = _layer(h, mask, p['layers'][1])
    return _head(h, p['gf'].reshape(1, _D), p['bf'].reshape(1, _D),
                 p['Wc1'], p['bc1'].reshape(1, _D // 2),
                 p['Wc2'], p['bc2'].reshape(1, _NCLS))


# speedup vs baseline: 1.2304x; 1.0014x over previous
import functools
import math

import jax
import jax.numpy as jnp
from jax import lax
from jax.experimental import pallas as pl
from jax.experimental.pallas import tpu as pltpu
from jax.experimental.pallas import tpu_sc as plsc

_IMG = 224
_P = 16
_C = 3
_V = 30522
_L = 77
_D = 1024
_E = 8
_H = 8
_NL = 2
_NCLS = 10
_DFF = 2048
_G = _IMG // _P
_NPATCH = _G * _G
_S = _NPATCH + _L
_B = 2
_SP = 288
_T = _B * _SP
_DH = _D // _H
_CPP = _C * _P * _P

_GELU_C = 0.7978845608028654
_DFF2 = _DFF // 2

_NW = 32
_BPW = 8
_GROWS = _NW * _BPW


def _ln_f32(x, g, b):
    m = jnp.mean(x, axis=-1, keepdims=True)
    v = jnp.mean((x - m) * (x - m), axis=-1, keepdims=True)
    return (x - m) * lax.rsqrt(v + 1e-5) * g + b


def _gelu(x):
    x3 = x * x * x
    return 0.5 * x * (1.0 + jnp.tanh(_GELU_C * (x + 0.044715 * x3)))


def _sc_gather(table, idx):
    mesh = plsc.VectorSubcoreMesh(core_axis_name="c", subcore_axis_name="s")

    @functools.partial(
        pl.kernel,
        mesh=mesh,
        out_type=jax.ShapeDtypeStruct((_GROWS, _D), jnp.float32),
        scratch_types=[
            pltpu.VMEM((_BPW,), jnp.int32),
            pltpu.VMEM((_BPW, _D), jnp.float32),
            pltpu.SemaphoreType.DMA,
        ],
    )
    def k(table_hbm, idx_hbm, out_hbm, idx_v, rows_v, sem):
        wid = lax.axis_index("s") * 2 + lax.axis_index("c")
        base = wid * _BPW
        pltpu.sync_copy(idx_hbm.at[pl.ds(base, _BPW)], idx_v)
        pltpu.async_copy(table_hbm.at[idx_v], rows_v, sem).wait()
        pltpu.sync_copy(rows_v, out_hbm.at[pl.ds(base, _BPW)])

    return k(table, idx)


def _embed_body(pat_ref, wp_ref, bp_ref, pos_img_ref, pos_txt_ref,
                mod0_ref, mod1_ref, txt_ref, out_ref):
    wp = wp_ref[...]
    zpad = jnp.zeros((_SP - _S, _D), jnp.float32)
    parts = []
    for b in range(_B):
        img = jnp.dot(pat_ref[b * _NPATCH:(b + 1) * _NPATCH, :], wp,
                      preferred_element_type=jnp.float32)
        img = img + bp_ref[...] + pos_img_ref[...] + mod0_ref[...]
        txt = txt_ref[b * _L:(b + 1) * _L, :] + pos_txt_ref[...] + mod1_ref[...]
        parts += [img, txt, zpad]
    out_ref[...] = jnp.concatenate(parts, axis=0)


def _embed(patches, wp, bp, pos_img, pos_txt, mod0, mod1, txt):
    return pl.pallas_call(
        _embed_body,
        out_shape=jax.ShapeDtypeStruct((_T, _D), jnp.float32),
    )(patches, wp, bp, pos_img, pos_txt, mod0, mod1, txt)


def _layer_body(with_head, *refs):
    if with_head:
        (h_ref, mask_ref, g1_ref, b1n_ref, wqkv_ref, bqkv_ref, wo_ref, bo_ref,
         g2_ref, b2n_ref, wr_ref, br_ref, w1_ref, b1_ref, w2_ref, b2_ref,
         gf_ref, bf_ref, wc1_ref, bc1_ref, wc2_ref, bc2_ref,
         out_ref, lg_ref, tln_ref, gates_ref) = refs
    else:
        (h_ref, mask_ref, g1_ref, b1n_ref, wqkv_ref, bqkv_ref, wo_ref, bo_ref,
         g2_ref, b2n_ref, wr_ref, br_ref, w1_ref, b1_ref, w2_ref, b2_ref,
         out_ref, tln_ref, gates_ref) = refs
    o_ref = tln_ref
    e = pl.program_id(0)
    j = pl.program_id(1)

    @pl.when((e == 0) & (j == 0))
    def _attn_and_router():
        ln = _ln_f32(h_ref[...], g1_ref[...], b1n_ref[...])
        scale = 1.0 / math.sqrt(_DH)
        for b in range(_B):
            lb = ln[b * _SP:(b + 1) * _SP, :]
            bias = (1.0 - mask_ref[b:b + 1, :]) * (-1e9)
            for hh in range(_H):
                wq = wqkv_ref[:, hh * _DH:(hh + 1) * _DH]
                wk = wqkv_ref[:, _D + hh * _DH:_D + (hh + 1) * _DH]
                wv = wqkv_ref[:, 2 * _D + hh * _DH:2 * _D + (hh + 1) * _DH]
                q = jnp.dot(lb, wq, preferred_element_type=jnp.float32)
                q = q + bqkv_ref[:, hh * _DH:(hh + 1) * _DH]
                k = jnp.dot(lb, wk, preferred_element_type=jnp.float32)
                k = k + bqkv_ref[:, _D + hh * _DH:_D + (hh + 1) * _DH]
                v = jnp.dot(lb, wv, preferred_element_type=jnp.float32)
                v = v + bqkv_ref[:, 2 * _D + hh * _DH:2 * _D + (hh + 1) * _DH]
                att = lax.dot_general(q, k, (((1,), (1,)), ((), ())),
                                      preferred_element_type=jnp.float32)
                att = att * scale + bias
                att = att - jnp.max(att, axis=-1, keepdims=True)
                pr = jnp.exp(att)
                pr = pr / jnp.sum(pr, axis=-1, keepdims=True)
                o_ref[b * _SP:(b + 1) * _SP, hh * _DH:(hh + 1) * _DH] = (
                    jnp.dot(pr, v, preferred_element_type=jnp.float32))
        out = jnp.dot(o_ref[...], wo_ref[...], preferred_element_type=jnp.float32)
        xa = h_ref[...] + out + bo_ref[...]
        out_ref[...] = xa

        ln2 = _ln_f32(xa, g2_ref[...], b2n_ref[...])
        tln_ref[...] = ln2
        logits = jnp.dot(ln2, wr_ref[...], preferred_element_type=jnp.float32)
        logits = logits + br_ref[...]
        lanes = lax.broadcasted_iota(jnp.int32, (_T, _E), 1)
        m1 = jnp.max(logits, axis=-1, keepdims=True)
        a1 = jnp.min(jnp.where(logits == m1, lanes, _E), axis=-1, keepdims=True)
        l2 = jnp.where(lanes == a1, -jnp.inf, logits)
        m2 = jnp.max(l2, axis=-1, keepdims=True)
        a2 = jnp.min(jnp.where(l2 == m2, lanes, _E), axis=-1, keepdims=True)
        e2 = jnp.exp(m2 - m1)
        w1g = 1.0 / (1.0 + e2)
        w2g = e2 / (1.0 + e2)
        gates_ref[...] = (jnp.where(lanes == a1, w1g, 0.0)
                          + jnp.where(lanes == a2, w2g, 0.0))

    hmid = jnp.dot(tln_ref[...], w1_ref[0], preferred_element_type=jnp.float32,
                   precision=lax.Precision.DEFAULT)
    hmid = _gelu(hmid + b1_ref[0])
    contrib = jnp.dot(hmid, w2_ref[0], preferred_element_type=jnp.float32,
                      precision=lax.Precision.DEFAULT)

    onehot = (lax.broadcasted_iota(jnp.int32, (_E, 1), 0) == e).astype(jnp.float32)
    gcol = jnp.dot(gates_ref[...], onehot, preferred_element_type=jnp.float32)

    @pl.when(j == 0)
    def _bias2():
        out_ref[...] += gcol * b2_ref[0]

    out_ref[...] += gcol * contrib

    if with_head:
        @pl.when((e == _E - 1) & (j == 1))
        def _do_head():
            lnf = _ln_f32(out_ref[...], gf_ref[...], bf_ref[...])
            riota = lax.broadcasted_iota(jnp.int32, (_SP, 1), 0)
            wrow = jnp.where(riota < _S, 1.0 / _S, 0.0)
            fvs = []
            for b in range(_B):
                fvs.append(jnp.sum(lnf[b * _SP:(b + 1) * _SP, :] * wrow,
                                   axis=0, keepdims=True))
            fv = jnp.concatenate(fvs, axis=0)
            hcl = jnp.dot(fv, wc1_ref[...], preferred_element_type=jnp.float32)
            hcl = jnp.maximum(hcl + bc1_ref[...], 0.0)
            lg = jnp.dot(hcl, wc2_ref[...], preferred_element_type=jnp.float32)
            lg_ref[...] = lg + bc2_ref[...]


def _layer(h, mask, lp, head_params=None):
    const2 = lambda e, j: (0, 0)
    in_specs = [
        pl.BlockSpec((_T, _D), const2),
        pl.BlockSpec((_B, _SP), const2),
        pl.BlockSpec((1, _D), const2),
        pl.BlockSpec((1, _D), const2),
        pl.BlockSpec((_D, 3 * _D), const2),
        pl.BlockSpec((1, 3 * _D), const2),
        pl.BlockSpec((_D, _D), const2),
        pl.BlockSpec((1, _D), const2),
        pl.BlockSpec((1, _D), const2),
        pl.BlockSpec((1, _D), const2),
        pl.BlockSpec((_D, _E), const2),
        pl.BlockSpec((1, _E), const2),
        pl.BlockSpec((1, _D, _DFF2), lambda e, j: (e, 0, j)),
        pl.BlockSpec((1, 1, _DFF2), lambda e, j: (e, 0, j)),
        pl.BlockSpec((1, _DFF2, _D), lambda e, j: (e, j, 0)),
        pl.BlockSpec((1, 1, _D), lambda e, j: (e, 0, 0)),
    ]
    args = [h, mask, lp['g1'].reshape(1, _D), lp['b1n'].reshape(1, _D),
            lp['Wqkv'], lp['bqkv'].reshape(1, 3 * _D), lp['Wo'],
            lp['bo'].reshape(1, _D), lp['g2'].reshape(1, _D),
            lp['b2n'].reshape(1, _D), lp['Wr'], lp['br'].reshape(1, _E),
            lp['W1'], lp['b1'].reshape(_E, 1, _DFF), lp['W2'],
            lp['b2'].reshape(_E, 1, _D)]
    with_head = head_params is not None
    if with_head:
        gf, bf, wc1, bc1, wc2, bc2 = head_params
        in_specs += [
            pl.BlockSpec((1, _D), const2),
            pl.BlockSpec((1, _D), const2),
            pl.BlockSpec((_D, _D // 2), const2),
            pl.BlockSpec((1, _D // 2), const2),
            pl.BlockSpec((_D // 2, _NCLS), const2),
            pl.BlockSpec((1, _NCLS), const2),
        ]
        args += [gf.reshape(1, _D), bf.reshape(1, _D), wc1,
                 bc1.reshape(1, _D // 2), wc2, bc2.reshape(1, _NCLS)]
        out_specs = [pl.BlockSpec((_T, _D), const2),
                     pl.BlockSpec((_B, _NCLS), const2)]
        out_shape = [jax.ShapeDtypeStruct((_T, _D), jnp.float32),
                     jax.ShapeDtypeStruct((_B, _NCLS), jnp.float32)]
    else:
        out_specs = pl.BlockSpec((_T, _D), const2)
        out_shape = jax.ShapeDtypeStruct((_T, _D), jnp.float32)

    return pl.pallas_call(
        functools.partial(_layer_body, with_head),
        grid=(_E, 2),
        in_specs=in_specs,
        out_specs=out_specs,
        out_shape=out_shape,
        scratch_shapes=[
            pltpu.VMEM((_T, _D), jnp.float32),
            pltpu.VMEM((_T, _E), jnp.float32),
        ],
    )(*args)


def _head_body(h_ref, gf_ref, bf_ref, wc1_ref, bc1_ref, wc2_ref, bc2_ref,
               out_ref):
    ln = _ln_f32(h_ref[...], gf_ref[...], bf_ref[...])
    riota = lax.broadcasted_iota(jnp.int32, (_SP, 1), 0)
    w = jnp.where(riota < _S, 1.0 / _S, 0.0)
    fvs = []
    for b in range(_B):
        fvs.append(jnp.sum(ln[b * _SP:(b + 1) * _SP, :] * w, axis=0,
                           keepdims=True))
    fv = jnp.concatenate(fvs, axis=0)
    hcl = jnp.dot(fv, wc1_ref[...], preferred_element_type=jnp.float32)
    hcl = jnp.maximum(hcl + bc1_ref[...], 0.0)
    lg = jnp.dot(hcl, wc2_ref[...], preferred_element_type=jnp.float32)
    out_ref[...] = lg + bc2_ref[...]


def _head(h, gf, bf, wc1, bc1, wc2, bc2):
    return pl.pallas_call(
        _head_body,
        out_shape=jax.ShapeDtypeStruct((_B, _NCLS), jnp.float32),
    )(h, gf, bf, wc1, bc1, wc2, bc2)


def kernel(images, input_ids, attention_mask, params):
    p = params
    patches = images.reshape(_B, _C, _G, _P, _G, _P)
    patches = patches.transpose(0, 2, 4, 1, 3, 5).reshape(_B * _NPATCH, _CPP)

    ids = input_ids.reshape(-1).astype(jnp.int32)
    ids = jnp.concatenate([ids, jnp.zeros((_GROWS - _B * _L,), jnp.int32)])
    txt_rows = _sc_gather(p['tok_emb'], ids)[:_B * _L]

    h = _embed(patches, p['Wp'], p['bp'].reshape(1, _D), p['pos_img'],
               p['pos_txt'], p['mod'][0:1], p['mod'][1:2], txt_rows)

    mask = jnp.concatenate(
        [jnp.ones((_B, _NPATCH), jnp.float32),
         attention_mask.astype(jnp.float32),
         jnp.zeros((_B, _SP - _S), jnp.float32)], axis=1)

    h = _layer(h, mask, p['layers'][0])
    h = _layer(h, mask, p['layers'][1])
    return _head(h, p['gf'].reshape(1, _D), p['bf'].reshape(1, _D),
                 p['Wc1'], p['bc1'].reshape(1, _D // 2),
                 p['Wc2'], p['bc2'].reshape(1, _NCLS))


# speedup vs baseline: 1.2326x; 1.0018x over previous
import functools
import math

import jax
import jax.numpy as jnp
from jax import lax
from jax.experimental import pallas as pl
from jax.experimental.pallas import tpu as pltpu
from jax.experimental.pallas import tpu_sc as plsc

_IMG = 224
_P = 16
_C = 3
_V = 30522
_L = 77
_D = 1024
_E = 8
_H = 8
_NL = 2
_NCLS = 10
_DFF = 2048
_G = _IMG // _P
_NPATCH = _G * _G
_S = _NPATCH + _L
_B = 2
_SP = 288
_T = _B * _SP
_DH = _D // _H
_CPP = _C * _P * _P

_GELU_C = 0.7978845608028654
_DFF2 = _DFF // 2

_NW = 32
_BPW = 8
_GROWS = _NW * _BPW


def _ln_f32(x, g, b):
    m = jnp.mean(x, axis=-1, keepdims=True)
    v = jnp.mean((x - m) * (x - m), axis=-1, keepdims=True)
    return (x - m) * lax.rsqrt(v + 1e-5) * g + b


def _gelu(x):
    x3 = x * x * x
    return 0.5 * x * (1.0 + jnp.tanh(_GELU_C * (x + 0.044715 * x3)))


def _sc_gather(table, idx):
    mesh = plsc.VectorSubcoreMesh(core_axis_name="c", subcore_axis_name="s")

    @functools.partial(
        pl.kernel,
        mesh=mesh,
        out_type=jax.ShapeDtypeStruct((_GROWS, _D), jnp.float32),
        scratch_types=[
            pltpu.VMEM((_BPW,), jnp.int32),
            pltpu.VMEM((_BPW, _D), jnp.float32),
            pltpu.SemaphoreType.DMA,
        ],
    )
    def k(table_hbm, idx_hbm, out_hbm, idx_v, rows_v, sem):
        wid = lax.axis_index("s") * 2 + lax.axis_index("c")
        base = wid * _BPW
        pltpu.sync_copy(idx_hbm.at[pl.ds(base, _BPW)], idx_v)
        pltpu.async_copy(table_hbm.at[idx_v], rows_v, sem).wait()
        pltpu.sync_copy(rows_v, out_hbm.at[pl.ds(base, _BPW)])

    return k(table, idx)


def _embed_body(pat_ref, wp_ref, bp_ref, pos_img_ref, pos_txt_ref,
                mod0_ref, mod1_ref, txt_ref, out_ref):
    wp = wp_ref[...]
    zpad = jnp.zeros((_SP - _S, _D), jnp.float32)
    parts = []
    for b in range(_B):
        img = jnp.dot(pat_ref[b * _NPATCH:(b + 1) * _NPATCH, :], wp,
                      preferred_element_type=jnp.float32)
        img = img + bp_ref[...] + pos_img_ref[...] + mod0_ref[...]
        txt = txt_ref[b * _L:(b + 1) * _L, :] + pos_txt_ref[...] + mod1_ref[...]
        parts += [img, txt, zpad]
    out_ref[...] = jnp.concatenate(parts, axis=0)


def _embed(patches, wp, bp, pos_img, pos_txt, mod0, mod1, txt):
    return pl.pallas_call(
        _embed_body,
        out_shape=jax.ShapeDtypeStruct((_T, _D), jnp.float32),
    )(patches, wp, bp, pos_img, pos_txt, mod0, mod1, txt)


def _layer_body(with_head, ffn_bf16, *refs):
    if with_head:
        (h_ref, mask_ref, g1_ref, b1n_ref, wqkv_ref, bqkv_ref, wo_ref, bo_ref,
         g2_ref, b2n_ref, wr_ref, br_ref, w1_ref, b1_ref, w2_ref, b2_ref,
         gf_ref, bf_ref, wc1_ref, bc1_ref, wc2_ref, bc2_ref,
         out_ref, lg_ref, tln_ref, gates_ref) = refs
    else:
        (h_ref, mask_ref, g1_ref, b1n_ref, wqkv_ref, bqkv_ref, wo_ref, bo_ref,
         g2_ref, b2n_ref, wr_ref, br_ref, w1_ref, b1_ref, w2_ref, b2_ref,
         out_ref, tln_ref, gates_ref) = refs
    o_ref = tln_ref
    e = pl.program_id(0)
    j = pl.program_id(1)

    @pl.when((e == 0) & (j == 0))
    def _attn_and_router():
        ln = _ln_f32(h_ref[...], g1_ref[...], b1n_ref[...])
        scale = 1.0 / math.sqrt(_DH)
        for b in range(_B):
            lb = ln[b * _SP:(b + 1) * _SP, :]
            bias = (1.0 - mask_ref[b:b + 1, :]) * (-1e9)
            for hh in range(_H):
                wq = wqkv_ref[:, hh * _DH:(hh + 1) * _DH]
                wk = wqkv_ref[:, _D + hh * _DH:_D + (hh + 1) * _DH]
                wv = wqkv_ref[:, 2 * _D + hh * _DH:2 * _D + (hh + 1) * _DH]
                q = jnp.dot(lb, wq, preferred_element_type=jnp.float32)
                q = q + bqkv_ref[:, hh * _DH:(hh + 1) * _DH]
                k = jnp.dot(lb, wk, preferred_element_type=jnp.float32)
                k = k + bqkv_ref[:, _D + hh * _DH:_D + (hh + 1) * _DH]
                v = jnp.dot(lb, wv, preferred_element_type=jnp.float32)
                v = v + bqkv_ref[:, 2 * _D + hh * _DH:2 * _D + (hh + 1) * _DH]
                att = lax.dot_general(q, k, (((1,), (1,)), ((), ())),
                                      preferred_element_type=jnp.float32)
                att = att * scale + bias
                att = att - jnp.max(att, axis=-1, keepdims=True)
                pr = jnp.exp(att)
                pr = pr / jnp.sum(pr, axis=-1, keepdims=True)
                o_ref[b * _SP:(b + 1) * _SP, hh * _DH:(hh + 1) * _DH] = (
                    jnp.dot(pr, v, preferred_element_type=jnp.float32))
        out = jnp.dot(o_ref[...], wo_ref[...], preferred_element_type=jnp.float32)
        xa = h_ref[...] + out + bo_ref[...]
        out_ref[...] = xa

        ln2 = _ln_f32(xa, g2_ref[...], b2n_ref[...])
        tln_ref[...] = ln2
        logits = jnp.dot(ln2, wr_ref[...], preferred_element_type=jnp.float32)
        logits = logits + br_ref[...]
        lanes = lax.broadcasted_iota(jnp.int32, (_T, _E), 1)
        m1 = jnp.max(logits, axis=-1, keepdims=True)
        a1 = jnp.min(jnp.where(logits == m1, lanes, _E), axis=-1, keepdims=True)
        l2 = jnp.where(lanes == a1, -jnp.inf, logits)
        m2 = jnp.max(l2, axis=-1, keepdims=True)
        a2 = jnp.min(jnp.where(l2 == m2, lanes, _E), axis=-1, keepdims=True)
        e2 = jnp.exp(m2 - m1)
        w1g = 1.0 / (1.0 + e2)
        w2g = e2 / (1.0 + e2)
        gates_ref[...] = (jnp.where(lanes == a1, w1g, 0.0)
                          + jnp.where(lanes == a2, w2g, 0.0))

    if ffn_bf16:
        hmid = jnp.dot(tln_ref[...].astype(jnp.bfloat16),
                       w1_ref[0].astype(jnp.bfloat16),
                       preferred_element_type=jnp.float32)
        hmid = _gelu(hmid + b1_ref[0])
        contrib = jnp.dot(hmid.astype(jnp.bfloat16),
                          w2_ref[0].astype(jnp.bfloat16),
                          preferred_element_type=jnp.float32)
    else:
        hmid = jnp.dot(tln_ref[...], w1_ref[0], preferred_element_type=jnp.float32)
        hmid = _gelu(hmid + b1_ref[0])
        contrib = jnp.dot(hmid, w2_ref[0], preferred_element_type=jnp.float32)

    onehot = (lax.broadcasted_iota(jnp.int32, (_E, 1), 0) == e).astype(jnp.float32)
    gcol = jnp.dot(gates_ref[...], onehot, preferred_element_type=jnp.float32)

    @pl.when(j == 0)
    def _bias2():
        out_ref[...] += gcol * b2_ref[0]

    out_ref[...] += gcol * contrib

    if with_head:
        @pl.when((e == _E - 1) & (j == 1))
        def _do_head():
            lnf = _ln_f32(out_ref[...], gf_ref[...], bf_ref[...])
            riota = lax.broadcasted_iota(jnp.int32, (_SP, 1), 0)
            wrow = jnp.where(riota < _S, 1.0 / _S, 0.0)
            fvs = []
            for b in range(_B):
                fvs.append(jnp.sum(lnf[b * _SP:(b + 1) * _SP, :] * wrow,
                                   axis=0, keepdims=True))
            fv = jnp.concatenate(fvs, axis=0)
            hcl = jnp.dot(fv, wc1_ref[...], preferred_element_type=jnp.float32)
            hcl = jnp.maximum(hcl + bc1_ref[...], 0.0)
            lg = jnp.dot(hcl, wc2_ref[...], preferred_element_type=jnp.float32)
            lg_ref[...] = lg + bc2_ref[...]


def _layer(h, mask, lp, ffn_bf16=False, head_params=None):
    const2 = lambda e, j: (0, 0)
    in_specs = [
        pl.BlockSpec((_T, _D), const2),
        pl.BlockSpec((_B, _SP), const2),
        pl.BlockSpec((1, _D), const2),
        pl.BlockSpec((1, _D), const2),
        pl.BlockSpec((_D, 3 * _D), const2),
        pl.BlockSpec((1, 3 * _D), const2),
        pl.BlockSpec((_D, _D), const2),
        pl.BlockSpec((1, _D), const2),
        pl.BlockSpec((1, _D), const2),
        pl.BlockSpec((1, _D), const2),
        pl.BlockSpec((_D, _E), const2),
        pl.BlockSpec((1, _E), const2),
        pl.BlockSpec((1, _D, _DFF2), lambda e, j: (e, 0, j)),
        pl.BlockSpec((1, 1, _DFF2), lambda e, j: (e, 0, j)),
        pl.BlockSpec((1, _DFF2, _D), lambda e, j: (e, j, 0)),
        pl.BlockSpec((1, 1, _D), lambda e, j: (e, 0, 0)),
    ]
    args = [h, mask, lp['g1'].reshape(1, _D), lp['b1n'].reshape(1, _D),
            lp['Wqkv'], lp['bqkv'].reshape(1, 3 * _D), lp['Wo'],
            lp['bo'].reshape(1, _D), lp['g2'].reshape(1, _D),
            lp['b2n'].reshape(1, _D), lp['Wr'], lp['br'].reshape(1, _E),
            lp['W1'], lp['b1'].reshape(_E, 1, _DFF), lp['W2'],
            lp['b2'].reshape(_E, 1, _D)]
    with_head = head_params is not None
    if with_head:
        gf, bf, wc1, bc1, wc2, bc2 = head_params
        in_specs += [
            pl.BlockSpec((1, _D), const2),
            pl.BlockSpec((1, _D), const2),
            pl.BlockSpec((_D, _D // 2), const2),
            pl.BlockSpec((1, _D // 2), const2),
            pl.BlockSpec((_D // 2, _NCLS), const2),
            pl.BlockSpec((1, _NCLS), const2),
        ]
        args += [gf.reshape(1, _D), bf.reshape(1, _D), wc1,
                 bc1.reshape(1, _D // 2), wc2, bc2.reshape(1, _NCLS)]
        out_specs = [pl.BlockSpec((_T, _D), const2),
                     pl.BlockSpec((_B, _NCLS), const2)]
        out_shape = [jax.ShapeDtypeStruct((_T, _D), jnp.float32),
                     jax.ShapeDtypeStruct((_B, _NCLS), jnp.float32)]
    else:
        out_specs = pl.BlockSpec((_T, _D), const2)
        out_shape = jax.ShapeDtypeStruct((_T, _D), jnp.float32)

    return pl.pallas_call(
        functools.partial(_layer_body, with_head, ffn_bf16),
        grid=(_E, 2),
        in_specs=in_specs,
        out_specs=out_specs,
        out_shape=out_shape,
        scratch_shapes=[
            pltpu.VMEM((_T, _D), jnp.float32),
            pltpu.VMEM((_T, _E), jnp.float32),
        ],
    )(*args)


def _head_body(h_ref, gf_ref, bf_ref, wc1_ref, bc1_ref, wc2_ref, bc2_ref,
               out_ref):
    ln = _ln_f32(h_ref[...], gf_ref[...], bf_ref[...])
    riota = lax.broadcasted_iota(jnp.int32, (_SP, 1), 0)
    w = jnp.where(riota < _S, 1.0 / _S, 0.0)
    fvs = []
    for b in range(_B):
        fvs.append(jnp.sum(ln[b * _SP:(b + 1) * _SP, :] * w, axis=0,
                           keepdims=True))
    fv = jnp.concatenate(fvs, axis=0)
    hcl = jnp.dot(fv, wc1_ref[...], preferred_element_type=jnp.float32)
    hcl = jnp.maximum(hcl + bc1_ref[...], 0.0)
    lg = jnp.dot(hcl, wc2_ref[...], preferred_element_type=jnp.float32)
    out_ref[...] = lg + bc2_ref[...]


def _head(h, gf, bf, wc1, bc1, wc2, bc2):
    return pl.pallas_call(
        _head_body,
        out_shape=jax.ShapeDtypeStruct((_B, _NCLS), jnp.float32),
    )(h, gf, bf, wc1, bc1, wc2, bc2)


def kernel(images, input_ids, attention_mask, params):
    p = params
    patches = images.reshape(_B, _C, _G, _P, _G, _P)
    patches = patches.transpose(0, 2, 4, 1, 3, 5).reshape(_B * _NPATCH, _CPP)

    ids = input_ids.reshape(-1).astype(jnp.int32)
    ids = jnp.concatenate([ids, jnp.zeros((_GROWS - _B * _L,), jnp.int32)])
    txt_rows = _sc_gather(p['tok_emb'], ids)[:_B * _L]

    h = _embed(patches, p['Wp'], p['bp'].reshape(1, _D), p['pos_img'],
               p['pos_txt'], p['mod'][0:1], p['mod'][1:2], txt_rows)

    mask = jnp.concatenate(
        [jnp.ones((_B, _NPATCH), jnp.float32),
         attention_mask.astype(jnp.float32),
         jnp.zeros((_B, _SP - _S), jnp.float32)], axis=1)

    h = _layer(h, mask, p['layers'][0])
    h = _layer(h, mask, p['layers'][1], ffn_bf16=True)
    return _head(h, p['gf'].reshape(1, _D), p['bf'].reshape(1, _D),
                 p['Wc1'], p['bc1'].reshape(1, _D // 2),
                 p['Wc2'], p['bc2'].reshape(1, _NCLS))
